# baseline (speedup 1.0000x reference)
import functools

import jax
import jax.numpy as jnp
from jax.experimental import pallas as pl

_H = 16
_CHUNK = 256
_TOP_K = 5
_NEG = -1e9


def _mm_t_kernel(a_ref, w_ref, o_ref, *, cast_bf16):
    a = a_ref[...]
    w = w_ref[...]
    if cast_bf16:
        a = a.astype(jnp.bfloat16)
        w = w.astype(jnp.bfloat16)
    o_ref[...] = jax.lax.dot_general(
        a, w, (((1,), (1,)), ((), ())),
        preferred_element_type=jnp.float32)


def _matmul_t(a, w, bm, bn, cast_bf16=False):
    M, K = a.shape
    N = w.shape[0]
    return pl.pallas_call(
        functools.partial(_mm_t_kernel, cast_bf16=cast_bf16),
        grid=(M // bm, N // bn),
        in_specs=[
            pl.BlockSpec((bm, K), lambda i, j: (i, 0)),
            pl.BlockSpec((bn, K), lambda i, j: (j, 0)),
        ],
        out_specs=pl.BlockSpec((bm, bn), lambda i, j: (i, j)),
        out_shape=jax.ShapeDtypeStruct((M, N), jnp.float32),
    )(a, w)


def _routing_kernel(q_ref, k_ref, bias_ref, *, seq, d, scale):
    nc = seq // _CHUNK
    hn = _H * nc
    hd = d // _H
    Q = q_ref[...]
    K = k_ref[...]

    r2 = jax.lax.broadcasted_iota(jnp.int32, (seq, nc), 0) // _CHUNK
    c2 = jax.lax.broadcasted_iota(jnp.int32, (seq, nc), 1)
    S = (r2 == c2).astype(jnp.float32)
    cmT = jax.lax.dot_general(
        K, S, (((0,), (0,)), ((), ())),
        preferred_element_type=jnp.float32) * (1.0 / _CHUNK)

    rT = jax.lax.broadcasted_iota(jnp.int32, (nc, hn), 0)
    cT = jax.lax.broadcasted_iota(jnp.int32, (nc, hn), 1) % nc
    T = (rT == cT).astype(jnp.float32)
    CK = jnp.dot(cmT, T, preferred_element_type=jnp.float32)
    dr = jax.lax.broadcasted_iota(jnp.int32, (d, hn), 0) // hd
    dc = jax.lax.broadcasted_iota(jnp.int32, (d, hn), 1) // nc
    CK = jnp.where(dr == dc, CK, 0.0)

    sims = jnp.dot(Q, CK, preferred_element_type=jnp.float32) * scale

    lane = jax.lax.broadcasted_iota(jnp.int32, (seq, hn), 1) % nc
    rank = jnp.zeros((seq, hn), jnp.int32)
    pi = jax.lax.broadcasted_iota(jnp.int32, (hn, hn), 0)
    pj = jax.lax.broadcasted_iota(jnp.int32, (hn, hn), 1)
    same_grp = (pi // nc) == (pj // nc)
    for r in range(1, nc):
        M = jnp.logical_and(
            same_grp, (pi % nc) == ((pj % nc) + r) % nc
        ).astype(jnp.float32)
        B = jnp.dot(sims, M, preferred_element_type=jnp.float32)
        gt = B > sims
        tie = jnp.logical_and(B == sims, lane + r >= nc)
        rank = rank + jnp.logical_or(gt, tie).astype(jnp.int32)

    bias_ref[...] = jnp.where(rank < _TOP_K, 0.0, _NEG)


def _routing(q, k, scale):
    seq, d = q.shape
    nc = seq // _CHUNK
    kern = functools.partial(_routing_kernel, seq=seq, d=d, scale=scale)
    return pl.pallas_call(
        kern,
        grid=(1,),
        in_specs=[
            pl.BlockSpec((seq, d), lambda i: (0, 0)),
            pl.BlockSpec((seq, d), lambda i: (0, 0)),
        ],
        out_specs=pl.BlockSpec((seq, _H * nc), lambda i: (0, 0)),
        out_shape=jax.ShapeDtypeStruct((seq, _H * nc), jnp.float32),
    )(q, k)


def _attn_kernel(q_ref, k_ref, v_ref, b_ref, o_ref, *, seq, hd, hpp, scale):
    nc = seq // _CHUNK
    ri = jax.lax.broadcasted_iota(jnp.int32, (_CHUNK, _CHUNK), 0)
    ci = jax.lax.broadcasted_iota(jnp.int32, (_CHUNK, _CHUNK), 1)
    causal_bias = jnp.where(ri >= ci, 0.0, _NEG)

    head_outs = []
    for sh in range(hpp):
        c0 = sh * hd
        Qf = q_ref[:, c0:c0 + hd]
        Vf = v_ref[:, c0:c0 + hd]
        Qs = (Qf * scale).astype(jnp.bfloat16)
        K16 = k_ref[:, c0:c0 + hd].astype(jnp.bfloat16)
        V16 = Vf.astype(jnp.bfloat16)
        bias_h = b_ref[sh, :, :]
        mean_v = jnp.sum(Vf, axis=0, keepdims=True) * (1.0 / seq)

        out_chunks = []
        for cq in range(nc):
            q0 = cq * _CHUNK
            Qb = Qs[q0:q0 + _CHUNK, :]
            s_tiles = []
            b_cols = []
            mx = None
            for c in range(cq + 1):
                Kc = K16[c * _CHUNK:(c + 1) * _CHUNK, :]
                s_c = jax.lax.dot_general(
                    Qb, Kc, (((1,), (1,)), ((), ())),
                    preferred_element_type=jnp.float32)
                if c == cq:
                    s_c = s_c + causal_bias
                b_c = bias_h[q0:q0 + _CHUNK, c:c + 1]
                m_c = jnp.max(s_c, axis=1, keepdims=True) + b_c
                mx = m_c if mx is None else jnp.maximum(mx, m_c)
                s_tiles.append(s_c)
                b_cols.append(b_c)

            acc = None
            dn = None
            any_sel = None
            for c in range(cq + 1):
                off = mx - b_cols[c]
                p = jnp.exp(s_tiles[c] - off)
                pv = jax.lax.dot_general(
                    p.astype(jnp.bfloat16), V16[c * _CHUNK:(c + 1) * _CHUNK, :],
                    (((1,), (0,)), ((), ())),
                    preferred_element_type=jnp.float32)
                ds = jnp.sum(p, axis=1, keepdims=True)
                sel = b_cols[c] > -0.5
                acc = pv if acc is None else acc + pv
                dn = ds if dn is None else dn + ds
                any_sel = sel if any_sel is None else jnp.logical_or(
                    any_sel, sel)

            out = acc / dn
            out = jnp.where(any_sel, out,
                            jnp.broadcast_to(mean_v, (_CHUNK, hd)))
            out_chunks.append(out)
        head_outs.append(jnp.concatenate(out_chunks, axis=0))
    o_ref[...] = jnp.concatenate(head_outs, axis=1)


def _attention(q, k, v, bias3, scale):
    seq, d = q.shape
    nc = seq // _CHUNK
    hd = d // _H
    hpp = 2
    bw = hpp * hd
    kern = functools.partial(_attn_kernel, seq=seq, hd=hd, hpp=hpp,
                             scale=scale)
    return pl.pallas_call(
        kern,
        grid=(_H // hpp,),
        in_specs=[
            pl.BlockSpec((seq, bw), lambda h: (0, h)),
            pl.BlockSpec((seq, bw), lambda h: (0, h)),
            pl.BlockSpec((seq, bw), lambda h: (0, h)),
            pl.BlockSpec((hpp, seq, nc), lambda h: (h, 0, 0)),
        ],
        out_specs=pl.BlockSpec((seq, bw), lambda h: (0, h)),
        out_shape=jax.ShapeDtypeStruct((seq, d), jnp.float32),
    )(q, k, v, bias3)


def kernel(x, Wq, Wk, Wv, Wo):
    b, s, d = x.shape
    hd = d // _H
    nc = s // _CHUNK
    scale = hd ** -0.5
    x2d = x.reshape(b * s, d)
    q = _matmul_t(x2d, Wq, 256, 512)
    k = _matmul_t(x2d, Wk, 256, 512)
    v = _matmul_t(x2d, Wv, 256, 512, cast_bf16=True)
    bias = _routing(q, k, scale)
    bias3 = bias.reshape(s, _H, nc).transpose(1, 0, 2)
    attn = _attention(q, k, v, bias3, scale)
    out = _matmul_t(attn, Wo, 256, 512, cast_bf16=True)
    return out.reshape(b, s, d)
